# baseline (speedup 1.0000x reference)
import functools

import jax
import jax.numpy as jnp
from jax import lax
from jax.experimental import pallas as pl
from jax.experimental.pallas import tpu as pltpu
from jax.experimental.pallas import tpu_sc as plsc

N = 8192
D = 4
BIG = 128
HID = 16
K = 3
NG = 8

BR = 1024
BRM = 1024
CC = 512
NBLK = N // BR
NCHUNK = N // CC
FP = 16

_INF = float("inf")
_BIGI = 1e9


def _bn_kernel(x_ref, g_ref, b_ref, o_ref):
    x = x_ref[...]
    m = jnp.mean(x, axis=0, keepdims=True)
    d = x - m
    v = jnp.mean(d * d, axis=0, keepdims=True)
    h = d / jnp.sqrt(v + 1e-5) * g_ref[...] + b_ref[...]
    o_ref[...] = jnp.concatenate([h, jnp.zeros((N, FP - D), jnp.float32)], axis=1)


def _bn(x, gamma, beta):
    return pl.pallas_call(
        _bn_kernel,
        out_shape=jax.ShapeDtypeStruct((N, FP), jnp.float32),
    )(x, gamma.reshape(1, D), beta.reshape(1, D))


def _knn_kernel(nf, bounds_ref, xr_ref, chunks_ref, bc_ref, br_ref, idx_ref):
    pid = pl.program_id(0)
    c0 = bounds_ref[2 * pid]
    c1 = bounds_ref[2 * pid + 1]
    x_r = xr_ref[...]
    xr_n = x_r[:, :nf]
    sq_r = jnp.sum(xr_n * xr_n, axis=1, keepdims=True)
    b_r = br_ref[...]

    colid0 = lax.broadcasted_iota(jnp.int32, (BR, CC), 1).astype(jnp.float32)

    def body(c, carry):
        v1, i1, v2, i2, v3, i3 = carry
        ch = chunks_ref[pl.ds(c, 1)][0]
        b_c = bc_ref[pl.ds(c, 1)][0]
        xy = lax.dot_general(x_r, ch, (((1,), (0,)), ((), ())),
                             precision=lax.Precision.DEFAULT)
        ch_n = ch[:nf, :]
        sq_c = jnp.sum(ch_n * ch_n, axis=0, keepdims=True)
        dist = (sq_r + sq_c) - 2.0 * xy
        dist = jnp.where(b_r != b_c, _INF, dist)
        colid = jnp.float32(c * CC) + colid0

        def extract(d):
            m = jnp.min(d, axis=1, keepdims=True)
            cid = jnp.min(jnp.where(d == m, colid, _BIGI), axis=1,
                          keepdims=True)
            d = jnp.where(colid == cid, _INF, d)
            return m, cid, d

        def insert(v, i, v1, i1, v2, i2, v3, i3):
            b1 = v < v1
            b2 = v < v2
            b3 = v < v3
            nv1 = jnp.where(b1, v, v1)
            ni1 = jnp.where(b1, i, i1)
            nv2 = jnp.where(b1, v1, jnp.where(b2, v, v2))
            ni2 = jnp.where(b1, i1, jnp.where(b2, i, i2))
            nv3 = jnp.where(b2, v2, jnp.where(b3, v, v3))
            ni3 = jnp.where(b2, i2, jnp.where(b3, i, i3))
            return nv1, ni1, nv2, ni2, nv3, ni3

        for _ in range(K):
            m, cid, dist = extract(dist)
            v1, i1, v2, i2, v3, i3 = insert(m, cid, v1, i1, v2, i2, v3, i3)
        return v1, i1, v2, i2, v3, i3

    init = (jnp.full((BR, 1), _INF, jnp.float32),
            jnp.full((BR, 1), _BIGI, jnp.float32),
            jnp.full((BR, 1), _INF, jnp.float32),
            jnp.full((BR, 1), _BIGI, jnp.float32),
            jnp.full((BR, 1), _INF, jnp.float32),
            jnp.full((BR, 1), _BIGI, jnp.float32))
    _, i1, _, i2, _, i3 = lax.fori_loop(c0, c1, body, init)
    idx = jnp.concatenate([i1, i2, i3], axis=1)
    idx = jnp.minimum(idx, jnp.float32(N - 1))
    idx_ref[...] = idx.astype(jnp.int32)


def _knn(h_pad, chunks, bchunks, brow, bounds, nf):
    return pl.pallas_call(
        functools.partial(_knn_kernel, nf),
        grid=(NBLK,),
        in_specs=[
            pl.BlockSpec(memory_space=pltpu.SMEM),
            pl.BlockSpec((BR, FP), lambda r: (r, 0)),
            pl.BlockSpec((NCHUNK, FP, CC), lambda r: (0, 0, 0)),
            pl.BlockSpec((NCHUNK, 1, CC), lambda r: (0, 0, 0)),
            pl.BlockSpec((BR, 1), lambda r: (r, 0)),
        ],
        out_specs=pl.BlockSpec((BR, K), lambda r: (r, 0)),
        out_shape=jax.ShapeDtypeStruct((N, K), jnp.int32),
    )(bounds, h_pad, chunks, bchunks, brow)


B_TOT = K * N
NW = 32
B_PER_W = B_TOT // NW


def _sc_gather_call(table, flat_idx):
    mesh = plsc.VectorSubcoreMesh(core_axis_name="c", subcore_axis_name="s")

    @functools.partial(
        pl.kernel, mesh=mesh,
        compiler_params=pltpu.CompilerParams(use_tc_tiling_on_sc=False),
        out_type=jax.ShapeDtypeStruct((B_TOT, FP), jnp.float32),
        scratch_types=[
            pltpu.VMEM((B_PER_W,), jnp.int32),
            pltpu.VMEM((B_PER_W, FP), jnp.float32),
            pltpu.SemaphoreType.DMA,
        ],
    )
    def gk(table_hbm, idx_hbm, out_hbm, idx_v, rows_v, sem):
        wid = lax.axis_index("s") * 2 + lax.axis_index("c")
        base = wid * B_PER_W
        pltpu.sync_copy(idx_hbm.at[pl.ds(base, B_PER_W)], idx_v)
        pltpu.async_copy(table_hbm.at[idx_v], rows_v, sem).wait()
        pltpu.sync_copy(rows_v, out_hbm.at[pl.ds(base, B_PER_W)])

    return gk(table, flat_idx)


def _gather(table, idx):
    flat = jnp.transpose(idx).reshape(B_TOT)
    rows = _sc_gather_call(table, flat)
    return rows.reshape(K, N, FP)


def _enc_kernel(xi_ref, xj_ref, w1_ref, b1_ref, w2_ref, b2_ref, w3_ref,
                b3_ref, o_ref):
    xi = xi_ref[:, :D]
    w1, b1 = w1_ref[...], b1_ref[...]
    w2, b2 = w2_ref[...], b2_ref[...]
    w3, b3 = w3_ref[...], b3_ref[...]
    acc = jnp.zeros((BRM, HID), jnp.float32)
    for j in range(K):
        xj = xj_ref[j][:, :D]
        cat = jnp.concatenate([xi, xj - xi], axis=1)
        t = jax.nn.relu(lax.dot_general(cat, w1, (((1,), (0,)), ((), ())),
                                        precision=lax.Precision.DEFAULT) + b1)
        t = jax.nn.relu(lax.dot_general(t, w2, (((1,), (0,)), ((), ())),
                                        precision=lax.Precision.DEFAULT) + b2)
        t = jax.nn.relu(lax.dot_general(t, w3, (((1,), (0,)), ((), ())),
                                        precision=lax.Precision.DEFAULT) + b3)
        acc = acc + t
    o_ref[...] = acc * jnp.float32(1.0 / K)


def _enc(h_pad, xj, eW1, eb1, eW2, eb2, eW3, eb3):
    return pl.pallas_call(
        _enc_kernel,
        grid=(N // BRM,),
        in_specs=[
            pl.BlockSpec((BRM, FP), lambda r: (r, 0)),
            pl.BlockSpec((K, BRM, FP), lambda r: (0, r, 0)),
            pl.BlockSpec((2 * D, BIG), lambda r: (0, 0)),
            pl.BlockSpec((1, BIG), lambda r: (0, 0)),
            pl.BlockSpec((BIG, BIG), lambda r: (0, 0)),
            pl.BlockSpec((1, BIG), lambda r: (0, 0)),
            pl.BlockSpec((BIG, HID), lambda r: (0, 0)),
            pl.BlockSpec((1, HID), lambda r: (0, 0)),
        ],
        out_specs=pl.BlockSpec((BRM, HID), lambda r: (r, 0)),
        out_shape=jax.ShapeDtypeStruct((N, HID), jnp.float32),
    )(h_pad, xj, eW1, eb1.reshape(1, BIG), eW2, eb2.reshape(1, BIG),
      eW3, eb3.reshape(1, HID))


def _dec_kernel(xi_ref, xj_ref, w1_ref, b1_ref, w2_ref, b2_ref, w3_ref,
                b3_ref, o_ref):
    xi = xi_ref[...]
    w1, b1 = w1_ref[...], b1_ref[...]
    w2, b2 = w2_ref[...], b2_ref[...]
    w3, b3 = w3_ref[...], b3_ref[...]
    acc = jnp.zeros((BRM, BIG), jnp.float32)
    for j in range(K):
        xj = xj_ref[j]
        cat = jnp.concatenate([xi, xj - xi], axis=1)
        t = jax.nn.relu(lax.dot_general(cat, w1, (((1,), (0,)), ((), ())),
                                        precision=lax.Precision.DEFAULT) + b1)
        t = jax.nn.relu(lax.dot_general(t, w2, (((1,), (0,)), ((), ())),
                                        precision=lax.Precision.DEFAULT) + b2)
        acc = acc + t
    o_ref[...] = lax.dot_general(acc * jnp.float32(1.0 / K), w3,
                                 (((1,), (0,)), ((), ())),
                                 precision=lax.Precision.DEFAULT) + b3


def _dec(h1, xj, dW1, db1, dW2, db2, dW3, db3):
    return pl.pallas_call(
        _dec_kernel,
        grid=(N // BRM,),
        in_specs=[
            pl.BlockSpec((BRM, HID), lambda r: (r, 0)),
            pl.BlockSpec((K, BRM, HID), lambda r: (0, r, 0)),
            pl.BlockSpec((2 * HID, BIG), lambda r: (0, 0)),
            pl.BlockSpec((1, BIG), lambda r: (0, 0)),
            pl.BlockSpec((BIG, BIG), lambda r: (0, 0)),
            pl.BlockSpec((1, BIG), lambda r: (0, 0)),
            pl.BlockSpec((BIG, D), lambda r: (0, 0)),
            pl.BlockSpec((1, D), lambda r: (0, 0)),
        ],
        out_specs=pl.BlockSpec((BRM, D), lambda r: (r, 0)),
        out_shape=jax.ShapeDtypeStruct((N, D), jnp.float32),
    )(h1, xj, dW1, db1.reshape(1, BIG), dW2, db2.reshape(1, BIG),
      dW3, db3.reshape(1, D))


def _chunked(h_pad):
    return jnp.transpose(h_pad).reshape(FP, NCHUNK, CC).transpose(1, 0, 2)


def _bounds(batch):
    g = jnp.arange(NG, dtype=jnp.int32)
    starts = jnp.searchsorted(batch, g, side="left").astype(jnp.int32)
    ends = jnp.searchsorted(batch, g, side="right").astype(jnp.int32)
    b2 = batch.reshape(NBLK, BR)
    lo = starts[b2[:, 0]]
    hi = ends[b2[:, -1]]
    c0 = lo // CC
    c1 = (hi + CC - 1) // CC
    return jnp.stack([c0, c1], axis=1).reshape(2 * NBLK).astype(jnp.int32)


def kernel(x, batch, bn_gamma, bn_beta, eW1, eb1, eW2, eb2, eW3, eb3,
           dW1, db1, dW2, db2, dW3, db3):
    bounds = _bounds(batch)
    bchunks = batch.reshape(NCHUNK, 1, CC)
    brow = batch.reshape(N, 1)

    h0 = _bn(x, bn_gamma, bn_beta)
    idx1 = _knn(h0, _chunked(h0), bchunks, brow, bounds, D)
    xj1 = _gather(h0, idx1)
    h1 = _enc(h0, xj1, eW1, eb1, eW2, eb2, eW3, eb3)
    idx2 = _knn(h1, _chunked(h1), bchunks, brow, bounds, HID)
    xj2 = _gather(h1, idx2)
    out = _dec(h1, xj2, dW1, db1, dW2, db2, dW3, db3)
    return out


# speedup vs baseline: 1.1105x; 1.1105x over previous
import functools

import jax
import jax.numpy as jnp
from jax import lax
from jax.experimental import pallas as pl
from jax.experimental.pallas import tpu as pltpu
from jax.experimental.pallas import tpu_sc as plsc

N = 8192
D = 4
BIG = 128
HID = 16
K = 3
NG = 8

BR = 512
BRM = 1024
CC = 512
NBLK = N // BR
NCHUNK = N // CC
FP = 16

_INF = float("inf")
_BIGI = 1e9


def _bn_kernel(x_ref, g_ref, b_ref, o_ref):
    x = x_ref[...]
    m = jnp.mean(x, axis=0, keepdims=True)
    d = x - m
    v = jnp.mean(d * d, axis=0, keepdims=True)
    h = d / jnp.sqrt(v + 1e-5) * g_ref[...] + b_ref[...]
    o_ref[...] = jnp.concatenate([h, jnp.zeros((N, FP - D), jnp.float32)], axis=1)


def _bn(x, gamma, beta):
    return pl.pallas_call(
        _bn_kernel,
        out_shape=jax.ShapeDtypeStruct((N, FP), jnp.float32),
    )(x, gamma.reshape(1, D), beta.reshape(1, D))


def _knn_kernel(nf, bounds_ref, xr_ref, chunks_ref, bc_ref, br_ref, idx_ref):
    pid = pl.program_id(0)
    c0 = bounds_ref[2 * pid]
    c1 = bounds_ref[2 * pid + 1]
    x_r = xr_ref[...]
    xr_n = x_r[:, :nf]
    sq_r = jnp.sum(xr_n * xr_n, axis=1, keepdims=True)
    b_r = br_ref[...]

    colid0 = lax.broadcasted_iota(jnp.int32, (BR, CC), 1).astype(jnp.float32)

    def body(c, carry):
        v1, i1, v2, i2, v3, i3 = carry
        ch = chunks_ref[pl.ds(c, 1)][0]
        b_c = bc_ref[pl.ds(c, 1)][0]
        xy = lax.dot_general(x_r, ch, (((1,), (0,)), ((), ())),
                             precision=lax.Precision.DEFAULT)
        ch_n = ch[:nf, :]
        sq_c = jnp.sum(ch_n * ch_n, axis=0, keepdims=True)
        dist = (sq_r + sq_c) - 2.0 * xy
        dist = jnp.where(b_r != b_c, _INF, dist)
        colid = jnp.float32(c * CC) + colid0

        def extract(d):
            m = jnp.min(d, axis=1, keepdims=True)
            cid = jnp.min(jnp.where(d == m, colid, _BIGI), axis=1,
                          keepdims=True)
            d = jnp.where(colid == cid, _INF, d)
            return m, cid, d

        def insert(v, i, v1, i1, v2, i2, v3, i3):
            b1 = v < v1
            b2 = v < v2
            b3 = v < v3
            nv1 = jnp.where(b1, v, v1)
            ni1 = jnp.where(b1, i, i1)
            nv2 = jnp.where(b1, v1, jnp.where(b2, v, v2))
            ni2 = jnp.where(b1, i1, jnp.where(b2, i, i2))
            nv3 = jnp.where(b2, v2, jnp.where(b3, v, v3))
            ni3 = jnp.where(b2, i2, jnp.where(b3, i, i3))
            return nv1, ni1, nv2, ni2, nv3, ni3

        for _ in range(K):
            m, cid, dist = extract(dist)
            v1, i1, v2, i2, v3, i3 = insert(m, cid, v1, i1, v2, i2, v3, i3)
        return v1, i1, v2, i2, v3, i3

    init = (jnp.full((BR, 1), _INF, jnp.float32),
            jnp.full((BR, 1), _BIGI, jnp.float32),
            jnp.full((BR, 1), _INF, jnp.float32),
            jnp.full((BR, 1), _BIGI, jnp.float32),
            jnp.full((BR, 1), _INF, jnp.float32),
            jnp.full((BR, 1), _BIGI, jnp.float32))
    _, i1, _, i2, _, i3 = lax.fori_loop(c0, c1, body, init)
    idx = jnp.concatenate([i1, i2, i3], axis=1)
    idx = jnp.minimum(idx, jnp.float32(N - 1))
    idx_ref[...] = idx.astype(jnp.int32)


def _knn(h_pad, chunks, bchunks, brow, bounds, nf):
    return pl.pallas_call(
        functools.partial(_knn_kernel, nf),
        grid=(NBLK,),
        in_specs=[
            pl.BlockSpec(memory_space=pltpu.SMEM),
            pl.BlockSpec((BR, FP), lambda r: (r, 0)),
            pl.BlockSpec((NCHUNK, FP, CC), lambda r: (0, 0, 0)),
            pl.BlockSpec((NCHUNK, 1, CC), lambda r: (0, 0, 0)),
            pl.BlockSpec((BR, 1), lambda r: (r, 0)),
        ],
        out_specs=pl.BlockSpec((BR, K), lambda r: (r, 0)),
        out_shape=jax.ShapeDtypeStruct((N, K), jnp.int32),
    )(bounds, h_pad, chunks, bchunks, brow)


B_TOT = K * N
NW = 32
B_PER_W = B_TOT // NW


def _sc_gather_call(table, flat_idx):
    mesh = plsc.VectorSubcoreMesh(core_axis_name="c", subcore_axis_name="s")

    @functools.partial(
        pl.kernel, mesh=mesh,
        compiler_params=pltpu.CompilerParams(use_tc_tiling_on_sc=False),
        out_type=jax.ShapeDtypeStruct((B_TOT, FP), jnp.float32),
        scratch_types=[
            pltpu.VMEM((B_PER_W,), jnp.int32),
            pltpu.VMEM((B_PER_W, FP), jnp.float32),
            pltpu.SemaphoreType.DMA,
        ],
    )
    def gk(table_hbm, idx_hbm, out_hbm, idx_v, rows_v, sem):
        wid = lax.axis_index("s") * 2 + lax.axis_index("c")
        base = wid * B_PER_W
        pltpu.sync_copy(idx_hbm.at[pl.ds(base, B_PER_W)], idx_v)
        pltpu.async_copy(table_hbm.at[idx_v], rows_v, sem).wait()
        pltpu.sync_copy(rows_v, out_hbm.at[pl.ds(base, B_PER_W)])

    return gk(table, flat_idx)


def _gather(table, idx):
    flat = jnp.transpose(idx).reshape(B_TOT)
    rows = _sc_gather_call(table, flat)
    return rows.reshape(K, N, FP)


def _enc_kernel(xi_ref, xj_ref, w1_ref, b1_ref, w2_ref, b2_ref, w3_ref,
                b3_ref, o_ref):
    xi = xi_ref[:, :D]
    w1, b1 = w1_ref[...], b1_ref[...]
    w2, b2 = w2_ref[...], b2_ref[...]
    w3, b3 = w3_ref[...], b3_ref[...]
    acc = jnp.zeros((BRM, HID), jnp.float32)
    for j in range(K):
        xj = xj_ref[j][:, :D]
        cat = jnp.concatenate([xi, xj - xi], axis=1)
        t = jax.nn.relu(lax.dot_general(cat, w1, (((1,), (0,)), ((), ())),
                                        precision=lax.Precision.DEFAULT) + b1)
        t = jax.nn.relu(lax.dot_general(t, w2, (((1,), (0,)), ((), ())),
                                        precision=lax.Precision.DEFAULT) + b2)
        t = jax.nn.relu(lax.dot_general(t, w3, (((1,), (0,)), ((), ())),
                                        precision=lax.Precision.DEFAULT) + b3)
        acc = acc + t
    o_ref[...] = acc * jnp.float32(1.0 / K)


def _enc(h_pad, xj, eW1, eb1, eW2, eb2, eW3, eb3):
    return pl.pallas_call(
        _enc_kernel,
        grid=(N // BRM,),
        in_specs=[
            pl.BlockSpec((BRM, FP), lambda r: (r, 0)),
            pl.BlockSpec((K, BRM, FP), lambda r: (0, r, 0)),
            pl.BlockSpec((2 * D, BIG), lambda r: (0, 0)),
            pl.BlockSpec((1, BIG), lambda r: (0, 0)),
            pl.BlockSpec((BIG, BIG), lambda r: (0, 0)),
            pl.BlockSpec((1, BIG), lambda r: (0, 0)),
            pl.BlockSpec((BIG, HID), lambda r: (0, 0)),
            pl.BlockSpec((1, HID), lambda r: (0, 0)),
        ],
        out_specs=pl.BlockSpec((BRM, HID), lambda r: (r, 0)),
        out_shape=jax.ShapeDtypeStruct((N, HID), jnp.float32),
    )(h_pad, xj, eW1, eb1.reshape(1, BIG), eW2, eb2.reshape(1, BIG),
      eW3, eb3.reshape(1, HID))


def _dec_kernel(xi_ref, xj_ref, w1_ref, b1_ref, w2_ref, b2_ref, w3_ref,
                b3_ref, o_ref):
    xi = xi_ref[...]
    w1, b1 = w1_ref[...], b1_ref[...]
    w2, b2 = w2_ref[...], b2_ref[...]
    w3, b3 = w3_ref[...], b3_ref[...]
    acc = jnp.zeros((BRM, BIG), jnp.float32)
    for j in range(K):
        xj = xj_ref[j]
        cat = jnp.concatenate([xi, xj - xi], axis=1)
        t = jax.nn.relu(lax.dot_general(cat, w1, (((1,), (0,)), ((), ())),
                                        precision=lax.Precision.DEFAULT) + b1)
        t = jax.nn.relu(lax.dot_general(t, w2, (((1,), (0,)), ((), ())),
                                        precision=lax.Precision.DEFAULT) + b2)
        acc = acc + t
    o_ref[...] = lax.dot_general(acc * jnp.float32(1.0 / K), w3,
                                 (((1,), (0,)), ((), ())),
                                 precision=lax.Precision.DEFAULT) + b3


def _dec(h1, xj, dW1, db1, dW2, db2, dW3, db3):
    return pl.pallas_call(
        _dec_kernel,
        grid=(N // BRM,),
        in_specs=[
            pl.BlockSpec((BRM, HID), lambda r: (r, 0)),
            pl.BlockSpec((K, BRM, HID), lambda r: (0, r, 0)),
            pl.BlockSpec((2 * HID, BIG), lambda r: (0, 0)),
            pl.BlockSpec((1, BIG), lambda r: (0, 0)),
            pl.BlockSpec((BIG, BIG), lambda r: (0, 0)),
            pl.BlockSpec((1, BIG), lambda r: (0, 0)),
            pl.BlockSpec((BIG, D), lambda r: (0, 0)),
            pl.BlockSpec((1, D), lambda r: (0, 0)),
        ],
        out_specs=pl.BlockSpec((BRM, D), lambda r: (r, 0)),
        out_shape=jax.ShapeDtypeStruct((N, D), jnp.float32),
    )(h1, xj, dW1, db1.reshape(1, BIG), dW2, db2.reshape(1, BIG),
      dW3, db3.reshape(1, D))


def _chunked(h_pad):
    return jnp.transpose(h_pad).reshape(FP, NCHUNK, CC).transpose(1, 0, 2)


def _bounds(batch):
    g = jnp.arange(NG, dtype=jnp.int32)
    starts = jnp.searchsorted(batch, g, side="left").astype(jnp.int32)
    ends = jnp.searchsorted(batch, g, side="right").astype(jnp.int32)
    b2 = batch.reshape(NBLK, BR)
    lo = starts[b2[:, 0]]
    hi = ends[b2[:, -1]]
    c0 = lo // CC
    c1 = (hi + CC - 1) // CC
    return jnp.stack([c0, c1], axis=1).reshape(2 * NBLK).astype(jnp.int32)


def kernel(x, batch, bn_gamma, bn_beta, eW1, eb1, eW2, eb2, eW3, eb3,
           dW1, db1, dW2, db2, dW3, db3):
    bounds = _bounds(batch)
    bchunks = batch.reshape(NCHUNK, 1, CC)
    brow = batch.reshape(N, 1)

    h0 = _bn(x, bn_gamma, bn_beta)
    idx1 = _knn(h0, _chunked(h0), bchunks, brow, bounds, D)
    xj1 = _gather(h0, idx1)
    h1 = _enc(h0, xj1, eW1, eb1, eW2, eb2, eW3, eb3)
    idx2 = _knn(h1, _chunked(h1), bchunks, brow, bounds, HID)
    xj2 = _gather(h1, idx2)
    out = _dec(h1, xj2, dW1, db1, dW2, db2, dW3, db3)
    return out


# speedup vs baseline: 1.1239x; 1.0120x over previous
import functools

import jax
import jax.numpy as jnp
from jax import lax
from jax.experimental import pallas as pl
from jax.experimental.pallas import tpu as pltpu
from jax.experimental.pallas import tpu_sc as plsc

N = 8192
D = 4
BIG = 128
HID = 16
K = 3
NG = 8

BR = 512
BRM = 1024
CC = 512
NBLK = N // BR
NCHUNK = N // CC
FP = 16

_INF = float("inf")
_BIGI = 1e9


def _bn_kernel(x_ref, g_ref, b_ref, o_ref, ot_ref):
    x = x_ref[...]
    m = jnp.mean(x, axis=0, keepdims=True)
    d = x - m
    v = jnp.mean(d * d, axis=0, keepdims=True)
    h = d / jnp.sqrt(v + 1e-5) * g_ref[...] + b_ref[...]
    hp = jnp.concatenate([h, jnp.zeros((N, FP - D), jnp.float32)], axis=1)
    o_ref[...] = hp
    for c in range(NCHUNK):
        ot_ref[c] = jnp.transpose(hp[c * CC:(c + 1) * CC, :])


def _bn(x, gamma, beta):
    return pl.pallas_call(
        _bn_kernel,
        out_shape=(jax.ShapeDtypeStruct((N, FP), jnp.float32),
                   jax.ShapeDtypeStruct((NCHUNK, FP, CC), jnp.float32)),
    )(x, gamma.reshape(1, D), beta.reshape(1, D))


def _knn_kernel(nf, bounds_ref, xr_ref, chunks_ref, bc_ref, br_ref, idx_ref):
    pid = pl.program_id(0)
    c0 = bounds_ref[2 * pid]
    c1 = bounds_ref[2 * pid + 1]
    x_r = xr_ref[...]
    xr_n = x_r[:, :nf]
    sq_r = jnp.sum(xr_n * xr_n, axis=1, keepdims=True)
    b_r = br_ref[...]

    colid0 = lax.broadcasted_iota(jnp.int32, (BR, CC), 1).astype(jnp.float32)

    def body(c, carry):
        v1, i1, v2, i2, v3, i3 = carry
        ch = chunks_ref[pl.ds(c, 1)][0]
        b_c = bc_ref[pl.ds(c, 1)][0]
        xy = lax.dot_general(x_r, ch, (((1,), (0,)), ((), ())),
                             precision=lax.Precision.DEFAULT)
        ch_n = ch[:nf, :]
        sq_c = jnp.sum(ch_n * ch_n, axis=0, keepdims=True)
        dist = (sq_r + sq_c) - 2.0 * xy
        dist = jnp.where(b_r != b_c, _INF, dist)
        colid = jnp.float32(c * CC) + colid0

        def extract(d):
            m = jnp.min(d, axis=1, keepdims=True)
            cid = jnp.min(jnp.where(d == m, colid, _BIGI), axis=1,
                          keepdims=True)
            d = jnp.where(colid == cid, _INF, d)
            return m, cid, d

        def insert(v, i, v1, i1, v2, i2, v3, i3):
            b1 = v < v1
            b2 = v < v2
            b3 = v < v3
            nv1 = jnp.where(b1, v, v1)
            ni1 = jnp.where(b1, i, i1)
            nv2 = jnp.where(b1, v1, jnp.where(b2, v, v2))
            ni2 = jnp.where(b1, i1, jnp.where(b2, i, i2))
            nv3 = jnp.where(b2, v2, jnp.where(b3, v, v3))
            ni3 = jnp.where(b2, i2, jnp.where(b3, i, i3))
            return nv1, ni1, nv2, ni2, nv3, ni3

        for _ in range(K):
            m, cid, dist = extract(dist)
            v1, i1, v2, i2, v3, i3 = insert(m, cid, v1, i1, v2, i2, v3, i3)
        return v1, i1, v2, i2, v3, i3

    init = (jnp.full((BR, 1), _INF, jnp.float32),
            jnp.full((BR, 1), _BIGI, jnp.float32),
            jnp.full((BR, 1), _INF, jnp.float32),
            jnp.full((BR, 1), _BIGI, jnp.float32),
            jnp.full((BR, 1), _INF, jnp.float32),
            jnp.full((BR, 1), _BIGI, jnp.float32))
    _, i1, _, i2, _, i3 = lax.fori_loop(c0, c1, body, init)
    idx = jnp.concatenate([i1, i2, i3], axis=1)
    idx = jnp.minimum(idx, jnp.float32(N - 1))
    idx_ref[...] = idx.astype(jnp.int32)


def _knn(h_pad, chunks, bchunks, brow, bounds, nf):
    return pl.pallas_call(
        functools.partial(_knn_kernel, nf),
        grid=(NBLK,),
        in_specs=[
            pl.BlockSpec(memory_space=pltpu.SMEM),
            pl.BlockSpec((BR, FP), lambda r: (r, 0)),
            pl.BlockSpec((NCHUNK, FP, CC), lambda r: (0, 0, 0)),
            pl.BlockSpec((NCHUNK, 1, CC), lambda r: (0, 0, 0)),
            pl.BlockSpec((BR, 1), lambda r: (r, 0)),
        ],
        out_specs=pl.BlockSpec((BR, K), lambda r: (r, 0)),
        out_shape=jax.ShapeDtypeStruct((N, K), jnp.int32),
    )(bounds, h_pad, chunks, bchunks, brow)


B_TOT = K * N
NW = 32
B_PER_W = B_TOT // NW


def _sc_gather_call(table, flat_idx):
    mesh = plsc.VectorSubcoreMesh(core_axis_name="c", subcore_axis_name="s")

    @functools.partial(
        pl.kernel, mesh=mesh,
        compiler_params=pltpu.CompilerParams(use_tc_tiling_on_sc=False),
        out_type=jax.ShapeDtypeStruct((B_TOT, FP), jnp.float32),
        scratch_types=[
            pltpu.VMEM((B_PER_W,), jnp.int32),
            pltpu.VMEM((B_PER_W, FP), jnp.float32),
            pltpu.SemaphoreType.DMA,
        ],
    )
    def gk(table_hbm, idx_hbm, out_hbm, idx_v, rows_v, sem):
        wid = lax.axis_index("s") * 2 + lax.axis_index("c")
        base = wid * B_PER_W
        pltpu.sync_copy(idx_hbm.at[pl.ds(base, B_PER_W)], idx_v)
        pltpu.async_copy(table_hbm.at[idx_v], rows_v, sem).wait()
        pltpu.sync_copy(rows_v, out_hbm.at[pl.ds(base, B_PER_W)])

    return gk(table, flat_idx)


def _gather(table, idx):
    flat = jnp.transpose(idx).reshape(B_TOT)
    rows = _sc_gather_call(table, flat)
    return rows.reshape(K, N, FP)


def _enc_kernel(xi_ref, xj_ref, w1_ref, b1_ref, w2_ref, b2_ref, w3_ref,
                b3_ref, o_ref, ot_ref):
    xi = xi_ref[:, :D]
    w1, b1 = w1_ref[...], b1_ref[...]
    w2, b2 = w2_ref[...], b2_ref[...]
    w3, b3 = w3_ref[...], b3_ref[...]
    acc = jnp.zeros((BRM, HID), jnp.float32)
    for j in range(K):
        xj = xj_ref[j][:, :D]
        cat = jnp.concatenate([xi, xj - xi], axis=1)
        t = jax.nn.relu(lax.dot_general(cat, w1, (((1,), (0,)), ((), ())),
                                        precision=lax.Precision.DEFAULT) + b1)
        t = jax.nn.relu(lax.dot_general(t, w2, (((1,), (0,)), ((), ())),
                                        precision=lax.Precision.DEFAULT) + b2)
        t = jax.nn.relu(lax.dot_general(t, w3, (((1,), (0,)), ((), ())),
                                        precision=lax.Precision.DEFAULT) + b3)
        acc = acc + t
    h = acc * jnp.float32(1.0 / K)
    o_ref[...] = h
    for s in range(BRM // CC):
        ot_ref[s] = jnp.transpose(h[s * CC:(s + 1) * CC, :])


def _enc(h_pad, xj, eW1, eb1, eW2, eb2, eW3, eb3):
    return pl.pallas_call(
        _enc_kernel,
        grid=(N // BRM,),
        in_specs=[
            pl.BlockSpec((BRM, FP), lambda r: (r, 0)),
            pl.BlockSpec((K, BRM, FP), lambda r: (0, r, 0)),
            pl.BlockSpec((2 * D, BIG), lambda r: (0, 0)),
            pl.BlockSpec((1, BIG), lambda r: (0, 0)),
            pl.BlockSpec((BIG, BIG), lambda r: (0, 0)),
            pl.BlockSpec((1, BIG), lambda r: (0, 0)),
            pl.BlockSpec((BIG, HID), lambda r: (0, 0)),
            pl.BlockSpec((1, HID), lambda r: (0, 0)),
        ],
        out_specs=(pl.BlockSpec((BRM, HID), lambda r: (r, 0)),
                   pl.BlockSpec((BRM // CC, FP, CC), lambda r: (r, 0, 0))),
        out_shape=(jax.ShapeDtypeStruct((N, HID), jnp.float32),
                   jax.ShapeDtypeStruct((NCHUNK, FP, CC), jnp.float32)),
    )(h_pad, xj, eW1, eb1.reshape(1, BIG), eW2, eb2.reshape(1, BIG),
      eW3, eb3.reshape(1, HID))


def _dec_kernel(xi_ref, xj_ref, w1_ref, b1_ref, w2_ref, b2_ref, w3_ref,
                b3_ref, o_ref):
    xi = xi_ref[...]
    w1, b1 = w1_ref[...], b1_ref[...]
    w2, b2 = w2_ref[...], b2_ref[...]
    w3, b3 = w3_ref[...], b3_ref[...]
    acc = jnp.zeros((BRM, BIG), jnp.float32)
    for j in range(K):
        xj = xj_ref[j]
        cat = jnp.concatenate([xi, xj - xi], axis=1)
        t = jax.nn.relu(lax.dot_general(cat, w1, (((1,), (0,)), ((), ())),
                                        precision=lax.Precision.DEFAULT) + b1)
        t = jax.nn.relu(lax.dot_general(t, w2, (((1,), (0,)), ((), ())),
                                        precision=lax.Precision.DEFAULT) + b2)
        acc = acc + t
    o_ref[...] = lax.dot_general(acc * jnp.float32(1.0 / K), w3,
                                 (((1,), (0,)), ((), ())),
                                 precision=lax.Precision.DEFAULT) + b3


def _dec(h1, xj, dW1, db1, dW2, db2, dW3, db3):
    return pl.pallas_call(
        _dec_kernel,
        grid=(N // BRM,),
        in_specs=[
            pl.BlockSpec((BRM, HID), lambda r: (r, 0)),
            pl.BlockSpec((K, BRM, HID), lambda r: (0, r, 0)),
            pl.BlockSpec((2 * HID, BIG), lambda r: (0, 0)),
            pl.BlockSpec((1, BIG), lambda r: (0, 0)),
            pl.BlockSpec((BIG, BIG), lambda r: (0, 0)),
            pl.BlockSpec((1, BIG), lambda r: (0, 0)),
            pl.BlockSpec((BIG, D), lambda r: (0, 0)),
            pl.BlockSpec((1, D), lambda r: (0, 0)),
        ],
        out_specs=pl.BlockSpec((BRM, D), lambda r: (r, 0)),
        out_shape=jax.ShapeDtypeStruct((N, D), jnp.float32),
    )(h1, xj, dW1, db1.reshape(1, BIG), dW2, db2.reshape(1, BIG),
      dW3, db3.reshape(1, D))


def _chunked(h_pad):
    return jnp.transpose(h_pad).reshape(FP, NCHUNK, CC).transpose(1, 0, 2)


def _bounds(batch):
    g = jnp.arange(NG, dtype=jnp.int32)
    starts = jnp.searchsorted(batch, g, side="left").astype(jnp.int32)
    ends = jnp.searchsorted(batch, g, side="right").astype(jnp.int32)
    b2 = batch.reshape(NBLK, BR)
    lo = starts[b2[:, 0]]
    hi = ends[b2[:, -1]]
    c0 = lo // CC
    c1 = (hi + CC - 1) // CC
    return jnp.stack([c0, c1], axis=1).reshape(2 * NBLK).astype(jnp.int32)


def kernel(x, batch, bn_gamma, bn_beta, eW1, eb1, eW2, eb2, eW3, eb3,
           dW1, db1, dW2, db2, dW3, db3):
    bounds = _bounds(batch)
    bchunks = batch.reshape(NCHUNK, 1, CC)
    brow = batch.reshape(N, 1)

    h0, ch0 = _bn(x, bn_gamma, bn_beta)
    idx1 = _knn(h0, ch0, bchunks, brow, bounds, D)
    xj1 = _gather(h0, idx1)
    h1, ch1 = _enc(h0, xj1, eW1, eb1, eW2, eb2, eW3, eb3)
    idx2 = _knn(h1, ch1, bchunks, brow, bounds, HID)
    xj2 = _gather(h1, idx2)
    out = _dec(h1, xj2, dW1, db1, dW2, db2, dW3, db3)
    return out


# speedup vs baseline: 1.1412x; 1.0154x over previous
import functools

import jax
import jax.numpy as jnp
from jax import lax
from jax.experimental import pallas as pl
from jax.experimental.pallas import tpu as pltpu
from jax.experimental.pallas import tpu_sc as plsc

N = 8192
D = 4
BIG = 128
HID = 16
K = 3
NG = 8

BR = 512
BRM = 2048
CC = 512
NBLK = N // BR
NCHUNK = N // CC
FP = 16

_INF = float("inf")
_BIGI = 1e9


def _bn_kernel(x_ref, g_ref, b_ref, o_ref, ot_ref):
    x = x_ref[...]
    m = jnp.mean(x, axis=0, keepdims=True)
    d = x - m
    v = jnp.mean(d * d, axis=0, keepdims=True)
    h = d / jnp.sqrt(v + 1e-5) * g_ref[...] + b_ref[...]
    hp = jnp.concatenate([h, jnp.zeros((N, FP - D), jnp.float32)], axis=1)
    o_ref[...] = hp
    for c in range(NCHUNK):
        ot_ref[c] = jnp.transpose(hp[c * CC:(c + 1) * CC, :])


def _bn(x, gamma, beta):
    return pl.pallas_call(
        _bn_kernel,
        out_shape=(jax.ShapeDtypeStruct((N, FP), jnp.float32),
                   jax.ShapeDtypeStruct((NCHUNK, FP, CC), jnp.float32)),
    )(x, gamma.reshape(1, D), beta.reshape(1, D))


def _knn_kernel(nf, bounds_ref, xr_ref, chunks_ref, bc_ref, br_ref, idx_ref):
    pid = pl.program_id(0)
    c0 = bounds_ref[2 * pid]
    c1 = bounds_ref[2 * pid + 1]
    x_r = xr_ref[...]
    xr_n = x_r[:, :nf]
    sq_r = jnp.sum(xr_n * xr_n, axis=1, keepdims=True)
    b_r = br_ref[...]

    colid0 = lax.broadcasted_iota(jnp.int32, (BR, CC), 1).astype(jnp.float32)

    def body(c, carry):
        v1, i1, v2, i2, v3, i3 = carry
        ch = chunks_ref[pl.ds(c, 1)][0]
        b_c = bc_ref[pl.ds(c, 1)][0]
        xy = lax.dot_general(x_r, ch, (((1,), (0,)), ((), ())),
                             precision=lax.Precision.DEFAULT)
        ch_n = ch[:nf, :]
        sq_c = jnp.sum(ch_n * ch_n, axis=0, keepdims=True)
        dist = (sq_r + sq_c) - 2.0 * xy
        dist = jnp.where(b_r != b_c, _INF, dist)
        colid = jnp.float32(c * CC) + colid0

        def extract(d):
            m = jnp.min(d, axis=1, keepdims=True)
            cid = jnp.min(jnp.where(d == m, colid, _BIGI), axis=1,
                          keepdims=True)
            d = jnp.where(colid == cid, _INF, d)
            return m, cid, d

        def insert(v, i, v1, i1, v2, i2, v3, i3):
            b1 = v < v1
            b2 = v < v2
            b3 = v < v3
            nv1 = jnp.where(b1, v, v1)
            ni1 = jnp.where(b1, i, i1)
            nv2 = jnp.where(b1, v1, jnp.where(b2, v, v2))
            ni2 = jnp.where(b1, i1, jnp.where(b2, i, i2))
            nv3 = jnp.where(b2, v2, jnp.where(b3, v, v3))
            ni3 = jnp.where(b2, i2, jnp.where(b3, i, i3))
            return nv1, ni1, nv2, ni2, nv3, ni3

        for _ in range(K):
            m, cid, dist = extract(dist)
            v1, i1, v2, i2, v3, i3 = insert(m, cid, v1, i1, v2, i2, v3, i3)
        return v1, i1, v2, i2, v3, i3

    init = (jnp.full((BR, 1), _INF, jnp.float32),
            jnp.full((BR, 1), _BIGI, jnp.float32),
            jnp.full((BR, 1), _INF, jnp.float32),
            jnp.full((BR, 1), _BIGI, jnp.float32),
            jnp.full((BR, 1), _INF, jnp.float32),
            jnp.full((BR, 1), _BIGI, jnp.float32))
    _, i1, _, i2, _, i3 = lax.fori_loop(c0, c1, body, init)
    idx = jnp.concatenate([i1, i2, i3], axis=1)
    idx = jnp.minimum(idx, jnp.float32(N - 1))
    idx_ref[...] = idx.astype(jnp.int32)


def _knn(h_pad, chunks, bchunks, brow, bounds, nf):
    return pl.pallas_call(
        functools.partial(_knn_kernel, nf),
        grid=(NBLK,),
        in_specs=[
            pl.BlockSpec(memory_space=pltpu.SMEM),
            pl.BlockSpec((BR, FP), lambda r: (r, 0)),
            pl.BlockSpec((NCHUNK, FP, CC), lambda r: (0, 0, 0)),
            pl.BlockSpec((NCHUNK, 1, CC), lambda r: (0, 0, 0)),
            pl.BlockSpec((BR, 1), lambda r: (r, 0)),
        ],
        out_specs=pl.BlockSpec((BR, K), lambda r: (r, 0)),
        out_shape=jax.ShapeDtypeStruct((N, K), jnp.int32),
    )(bounds, h_pad, chunks, bchunks, brow)


B_TOT = K * N
NW = 32
B_PER_W = B_TOT // NW


def _sc_gather_call(table, flat_idx):
    mesh = plsc.VectorSubcoreMesh(core_axis_name="c", subcore_axis_name="s")

    @functools.partial(
        pl.kernel, mesh=mesh,
        compiler_params=pltpu.CompilerParams(use_tc_tiling_on_sc=False),
        out_type=jax.ShapeDtypeStruct((B_TOT, FP), jnp.float32),
        scratch_types=[
            pltpu.VMEM((B_PER_W,), jnp.int32),
            pltpu.VMEM((B_PER_W, FP), jnp.float32),
            pltpu.SemaphoreType.DMA,
        ],
    )
    def gk(table_hbm, idx_hbm, out_hbm, idx_v, rows_v, sem):
        wid = lax.axis_index("s") * 2 + lax.axis_index("c")
        base = wid * B_PER_W
        pltpu.sync_copy(idx_hbm.at[pl.ds(base, B_PER_W)], idx_v)
        pltpu.async_copy(table_hbm.at[idx_v], rows_v, sem).wait()
        pltpu.sync_copy(rows_v, out_hbm.at[pl.ds(base, B_PER_W)])

    return gk(table, flat_idx)


def _gather(table, idx):
    flat = jnp.transpose(idx).reshape(B_TOT)
    rows = _sc_gather_call(table, flat)
    return rows.reshape(K, N, FP)


def _enc_kernel(xi_ref, xj_ref, w1_ref, b1_ref, w2_ref, b2_ref, w3_ref,
                b3_ref, o_ref, ot_ref):
    xi = xi_ref[:, :D]
    w1, b1 = w1_ref[...], b1_ref[...]
    w2, b2 = w2_ref[...], b2_ref[...]
    w3, b3 = w3_ref[...], b3_ref[...]
    acc = jnp.zeros((BRM, HID), jnp.float32)
    for j in range(K):
        xj = xj_ref[j][:, :D]
        cat = jnp.concatenate([xi, xj - xi], axis=1)
        t = jax.nn.relu(lax.dot_general(cat, w1, (((1,), (0,)), ((), ())),
                                        precision=lax.Precision.DEFAULT) + b1)
        t = jax.nn.relu(lax.dot_general(t, w2, (((1,), (0,)), ((), ())),
                                        precision=lax.Precision.DEFAULT) + b2)
        t = jax.nn.relu(lax.dot_general(t, w3, (((1,), (0,)), ((), ())),
                                        precision=lax.Precision.DEFAULT) + b3)
        acc = acc + t
    h = acc * jnp.float32(1.0 / K)
    o_ref[...] = h
    for s in range(BRM // CC):
        ot_ref[s] = jnp.transpose(h[s * CC:(s + 1) * CC, :])


def _enc(h_pad, xj, eW1, eb1, eW2, eb2, eW3, eb3):
    return pl.pallas_call(
        _enc_kernel,
        grid=(N // BRM,),
        in_specs=[
            pl.BlockSpec((BRM, FP), lambda r: (r, 0)),
            pl.BlockSpec((K, BRM, FP), lambda r: (0, r, 0)),
            pl.BlockSpec((2 * D, BIG), lambda r: (0, 0)),
            pl.BlockSpec((1, BIG), lambda r: (0, 0)),
            pl.BlockSpec((BIG, BIG), lambda r: (0, 0)),
            pl.BlockSpec((1, BIG), lambda r: (0, 0)),
            pl.BlockSpec((BIG, HID), lambda r: (0, 0)),
            pl.BlockSpec((1, HID), lambda r: (0, 0)),
        ],
        out_specs=(pl.BlockSpec((BRM, HID), lambda r: (r, 0)),
                   pl.BlockSpec((BRM // CC, FP, CC), lambda r: (r, 0, 0))),
        out_shape=(jax.ShapeDtypeStruct((N, HID), jnp.float32),
                   jax.ShapeDtypeStruct((NCHUNK, FP, CC), jnp.float32)),
    )(h_pad, xj, eW1, eb1.reshape(1, BIG), eW2, eb2.reshape(1, BIG),
      eW3, eb3.reshape(1, HID))


def _dec_kernel(xi_ref, xj_ref, w1_ref, b1_ref, w2_ref, b2_ref, w3_ref,
                b3_ref, o_ref):
    xi = xi_ref[...]
    w1, b1 = w1_ref[...], b1_ref[...]
    w2, b2 = w2_ref[...], b2_ref[...]
    w3, b3 = w3_ref[...], b3_ref[...]
    acc = jnp.zeros((BRM, BIG), jnp.float32)
    for j in range(K):
        xj = xj_ref[j]
        cat = jnp.concatenate([xi, xj - xi], axis=1)
        t = jax.nn.relu(lax.dot_general(cat, w1, (((1,), (0,)), ((), ())),
                                        precision=lax.Precision.DEFAULT) + b1)
        t = jax.nn.relu(lax.dot_general(t, w2, (((1,), (0,)), ((), ())),
                                        precision=lax.Precision.DEFAULT) + b2)
        acc = acc + t
    o_ref[...] = lax.dot_general(acc * jnp.float32(1.0 / K), w3,
                                 (((1,), (0,)), ((), ())),
                                 precision=lax.Precision.DEFAULT) + b3


def _dec(h1, xj, dW1, db1, dW2, db2, dW3, db3):
    return pl.pallas_call(
        _dec_kernel,
        grid=(N // BRM,),
        in_specs=[
            pl.BlockSpec((BRM, HID), lambda r: (r, 0)),
            pl.BlockSpec((K, BRM, HID), lambda r: (0, r, 0)),
            pl.BlockSpec((2 * HID, BIG), lambda r: (0, 0)),
            pl.BlockSpec((1, BIG), lambda r: (0, 0)),
            pl.BlockSpec((BIG, BIG), lambda r: (0, 0)),
            pl.BlockSpec((1, BIG), lambda r: (0, 0)),
            pl.BlockSpec((BIG, D), lambda r: (0, 0)),
            pl.BlockSpec((1, D), lambda r: (0, 0)),
        ],
        out_specs=pl.BlockSpec((BRM, D), lambda r: (r, 0)),
        out_shape=jax.ShapeDtypeStruct((N, D), jnp.float32),
    )(h1, xj, dW1, db1.reshape(1, BIG), dW2, db2.reshape(1, BIG),
      dW3, db3.reshape(1, D))


def _bounds(batch):
    g = jnp.arange(NG, dtype=jnp.int32)
    starts = jnp.searchsorted(batch, g, side="left").astype(jnp.int32)
    ends = jnp.searchsorted(batch, g, side="right").astype(jnp.int32)
    b2 = batch.reshape(NBLK, BR)
    lo = starts[b2[:, 0]]
    hi = ends[b2[:, -1]]
    c0 = lo // CC
    c1 = (hi + CC - 1) // CC
    return jnp.stack([c0, c1], axis=1).reshape(2 * NBLK).astype(jnp.int32)


def kernel(x, batch, bn_gamma, bn_beta, eW1, eb1, eW2, eb2, eW3, eb3,
           dW1, db1, dW2, db2, dW3, db3):
    bounds = _bounds(batch)
    bchunks = batch.reshape(NCHUNK, 1, CC)
    brow = batch.reshape(N, 1)

    h0, ch0 = _bn(x, bn_gamma, bn_beta)
    idx1 = _knn(h0, ch0, bchunks, brow, bounds, D)
    xj1 = _gather(h0, idx1)
    h1, ch1 = _enc(h0, xj1, eW1, eb1, eW2, eb2, eW3, eb3)
    idx2 = _knn(h1, ch1, bchunks, brow, bounds, HID)
    xj2 = _gather(h1, idx2)
    out = _dec(h1, xj2, dW1, db1, dW2, db2, dW3, db3)
    return out


# speedup vs baseline: 1.1424x; 1.0010x over previous
import functools

import jax
import jax.numpy as jnp
from jax import lax
from jax.experimental import pallas as pl
from jax.experimental.pallas import tpu as pltpu
from jax.experimental.pallas import tpu_sc as plsc

N = 8192
D = 4
BIG = 128
HID = 16
K = 3
NG = 8

BR = 512
BRM = 4096
CC = 512
NBLK = N // BR
NCHUNK = N // CC
FP = 16

_INF = float("inf")
_BIGI = 1e9


def _bn_kernel(x_ref, g_ref, b_ref, o_ref, ot_ref):
    x = x_ref[...]
    m = jnp.mean(x, axis=0, keepdims=True)
    d = x - m
    v = jnp.mean(d * d, axis=0, keepdims=True)
    h = d / jnp.sqrt(v + 1e-5) * g_ref[...] + b_ref[...]
    hp = jnp.concatenate([h, jnp.zeros((N, FP - D), jnp.float32)], axis=1)
    o_ref[...] = hp
    for c in range(NCHUNK):
        ot_ref[c] = jnp.transpose(hp[c * CC:(c + 1) * CC, :])


def _bn(x, gamma, beta):
    return pl.pallas_call(
        _bn_kernel,
        out_shape=(jax.ShapeDtypeStruct((N, FP), jnp.float32),
                   jax.ShapeDtypeStruct((NCHUNK, FP, CC), jnp.float32)),
    )(x, gamma.reshape(1, D), beta.reshape(1, D))


def _knn_kernel(nf, bounds_ref, xr_ref, chunks_ref, bc_ref, br_ref, idx_ref):
    pid = pl.program_id(0)
    c0 = bounds_ref[2 * pid]
    c1 = bounds_ref[2 * pid + 1]
    x_r = xr_ref[...]
    xr_n = x_r[:, :nf]
    sq_r = jnp.sum(xr_n * xr_n, axis=1, keepdims=True)
    b_r = br_ref[...]

    colid0 = lax.broadcasted_iota(jnp.int32, (BR, CC), 1).astype(jnp.float32)

    def body(c, carry):
        v1, i1, v2, i2, v3, i3 = carry
        ch = chunks_ref[pl.ds(c, 1)][0]
        b_c = bc_ref[pl.ds(c, 1)][0]
        xy = lax.dot_general(x_r, ch, (((1,), (0,)), ((), ())),
                             precision=lax.Precision.DEFAULT)
        ch_n = ch[:nf, :]
        sq_c = jnp.sum(ch_n * ch_n, axis=0, keepdims=True)
        dist = (sq_r + sq_c) - 2.0 * xy
        dist = jnp.where(b_r != b_c, _INF, dist)
        colid = jnp.float32(c * CC) + colid0

        def extract(d):
            m = jnp.min(d, axis=1, keepdims=True)
            cid = jnp.min(jnp.where(d == m, colid, _BIGI), axis=1,
                          keepdims=True)
            d = jnp.where(colid == cid, _INF, d)
            return m, cid, d

        def insert(v, i, v1, i1, v2, i2, v3, i3):
            b1 = v < v1
            b2 = v < v2
            b3 = v < v3
            nv1 = jnp.where(b1, v, v1)
            ni1 = jnp.where(b1, i, i1)
            nv2 = jnp.where(b1, v1, jnp.where(b2, v, v2))
            ni2 = jnp.where(b1, i1, jnp.where(b2, i, i2))
            nv3 = jnp.where(b2, v2, jnp.where(b3, v, v3))
            ni3 = jnp.where(b2, i2, jnp.where(b3, i, i3))
            return nv1, ni1, nv2, ni2, nv3, ni3

        for _ in range(K):
            m, cid, dist = extract(dist)
            v1, i1, v2, i2, v3, i3 = insert(m, cid, v1, i1, v2, i2, v3, i3)
        return v1, i1, v2, i2, v3, i3

    init = (jnp.full((BR, 1), _INF, jnp.float32),
            jnp.full((BR, 1), _BIGI, jnp.float32),
            jnp.full((BR, 1), _INF, jnp.float32),
            jnp.full((BR, 1), _BIGI, jnp.float32),
            jnp.full((BR, 1), _INF, jnp.float32),
            jnp.full((BR, 1), _BIGI, jnp.float32))
    _, i1, _, i2, _, i3 = lax.fori_loop(c0, c1, body, init)
    idx = jnp.concatenate([i1, i2, i3], axis=1)
    idx = jnp.minimum(idx, jnp.float32(N - 1))
    idx_ref[...] = idx.astype(jnp.int32)


def _knn(h_pad, chunks, bchunks, brow, bounds, nf):
    return pl.pallas_call(
        functools.partial(_knn_kernel, nf),
        grid=(NBLK,),
        in_specs=[
            pl.BlockSpec(memory_space=pltpu.SMEM),
            pl.BlockSpec((BR, FP), lambda r: (r, 0)),
            pl.BlockSpec((NCHUNK, FP, CC), lambda r: (0, 0, 0)),
            pl.BlockSpec((NCHUNK, 1, CC), lambda r: (0, 0, 0)),
            pl.BlockSpec((BR, 1), lambda r: (r, 0)),
        ],
        out_specs=pl.BlockSpec((BR, K), lambda r: (r, 0)),
        out_shape=jax.ShapeDtypeStruct((N, K), jnp.int32),
    )(bounds, h_pad, chunks, bchunks, brow)


B_TOT = K * N
NW = 32
B_PER_W = B_TOT // NW


def _sc_gather_call(table, flat_idx):
    mesh = plsc.VectorSubcoreMesh(core_axis_name="c", subcore_axis_name="s")

    @functools.partial(
        pl.kernel, mesh=mesh,
        compiler_params=pltpu.CompilerParams(use_tc_tiling_on_sc=False),
        out_type=jax.ShapeDtypeStruct((B_TOT, FP), jnp.float32),
        scratch_types=[
            pltpu.VMEM((B_PER_W,), jnp.int32),
            pltpu.VMEM((B_PER_W, FP), jnp.float32),
            pltpu.SemaphoreType.DMA,
        ],
    )
    def gk(table_hbm, idx_hbm, out_hbm, idx_v, rows_v, sem):
        wid = lax.axis_index("s") * 2 + lax.axis_index("c")
        base = wid * B_PER_W
        pltpu.sync_copy(idx_hbm.at[pl.ds(base, B_PER_W)], idx_v)
        pltpu.async_copy(table_hbm.at[idx_v], rows_v, sem).wait()
        pltpu.sync_copy(rows_v, out_hbm.at[pl.ds(base, B_PER_W)])

    return gk(table, flat_idx)


def _gather(table, idx):
    flat = jnp.transpose(idx).reshape(B_TOT)
    rows = _sc_gather_call(table, flat)
    return rows.reshape(K, N, FP)


def _enc_kernel(xi_ref, xj_ref, w1_ref, b1_ref, w2_ref, b2_ref, w3_ref,
                b3_ref, o_ref, ot_ref):
    xi = xi_ref[:, :D]
    w1, b1 = w1_ref[...], b1_ref[...]
    w2, b2 = w2_ref[...], b2_ref[...]
    w3, b3 = w3_ref[...], b3_ref[...]
    acc = jnp.zeros((BRM, HID), jnp.float32)
    for j in range(K):
        xj = xj_ref[j][:, :D]
        cat = jnp.concatenate([xi, xj - xi], axis=1)
        t = jax.nn.relu(lax.dot_general(cat, w1, (((1,), (0,)), ((), ())),
                                        precision=lax.Precision.DEFAULT) + b1)
        t = jax.nn.relu(lax.dot_general(t, w2, (((1,), (0,)), ((), ())),
                                        precision=lax.Precision.DEFAULT) + b2)
        t = jax.nn.relu(lax.dot_general(t, w3, (((1,), (0,)), ((), ())),
                                        precision=lax.Precision.DEFAULT) + b3)
        acc = acc + t
    h = acc * jnp.float32(1.0 / K)
    o_ref[...] = h
    for s in range(BRM // CC):
        ot_ref[s] = jnp.transpose(h[s * CC:(s + 1) * CC, :])


def _enc(h_pad, xj, eW1, eb1, eW2, eb2, eW3, eb3):
    return pl.pallas_call(
        _enc_kernel,
        grid=(N // BRM,),
        in_specs=[
            pl.BlockSpec((BRM, FP), lambda r: (r, 0)),
            pl.BlockSpec((K, BRM, FP), lambda r: (0, r, 0)),
            pl.BlockSpec((2 * D, BIG), lambda r: (0, 0)),
            pl.BlockSpec((1, BIG), lambda r: (0, 0)),
            pl.BlockSpec((BIG, BIG), lambda r: (0, 0)),
            pl.BlockSpec((1, BIG), lambda r: (0, 0)),
            pl.BlockSpec((BIG, HID), lambda r: (0, 0)),
            pl.BlockSpec((1, HID), lambda r: (0, 0)),
        ],
        out_specs=(pl.BlockSpec((BRM, HID), lambda r: (r, 0)),
                   pl.BlockSpec((BRM // CC, FP, CC), lambda r: (r, 0, 0))),
        out_shape=(jax.ShapeDtypeStruct((N, HID), jnp.float32),
                   jax.ShapeDtypeStruct((NCHUNK, FP, CC), jnp.float32)),
    )(h_pad, xj, eW1, eb1.reshape(1, BIG), eW2, eb2.reshape(1, BIG),
      eW3, eb3.reshape(1, HID))


def _dec_kernel(xi_ref, xj_ref, w1_ref, b1_ref, w2_ref, b2_ref, w3_ref,
                b3_ref, o_ref):
    xi = xi_ref[...]
    w1, b1 = w1_ref[...], b1_ref[...]
    w2, b2 = w2_ref[...], b2_ref[...]
    w3, b3 = w3_ref[...], b3_ref[...]
    acc = jnp.zeros((BRM, BIG), jnp.float32)
    for j in range(K):
        xj = xj_ref[j]
        cat = jnp.concatenate([xi, xj - xi], axis=1)
        t = jax.nn.relu(lax.dot_general(cat, w1, (((1,), (0,)), ((), ())),
                                        precision=lax.Precision.DEFAULT) + b1)
        t = jax.nn.relu(lax.dot_general(t, w2, (((1,), (0,)), ((), ())),
                                        precision=lax.Precision.DEFAULT) + b2)
        acc = acc + t
    o_ref[...] = lax.dot_general(acc * jnp.float32(1.0 / K), w3,
                                 (((1,), (0,)), ((), ())),
                                 precision=lax.Precision.DEFAULT) + b3


def _dec(h1, xj, dW1, db1, dW2, db2, dW3, db3):
    return pl.pallas_call(
        _dec_kernel,
        grid=(N // BRM,),
        in_specs=[
            pl.BlockSpec((BRM, HID), lambda r: (r, 0)),
            pl.BlockSpec((K, BRM, HID), lambda r: (0, r, 0)),
            pl.BlockSpec((2 * HID, BIG), lambda r: (0, 0)),
            pl.BlockSpec((1, BIG), lambda r: (0, 0)),
            pl.BlockSpec((BIG, BIG), lambda r: (0, 0)),
            pl.BlockSpec((1, BIG), lambda r: (0, 0)),
            pl.BlockSpec((BIG, D), lambda r: (0, 0)),
            pl.BlockSpec((1, D), lambda r: (0, 0)),
        ],
        out_specs=pl.BlockSpec((BRM, D), lambda r: (r, 0)),
        out_shape=jax.ShapeDtypeStruct((N, D), jnp.float32),
    )(h1, xj, dW1, db1.reshape(1, BIG), dW2, db2.reshape(1, BIG),
      dW3, db3.reshape(1, D))


def _bounds(batch):
    g = jnp.arange(NG, dtype=jnp.int32)
    starts = jnp.searchsorted(batch, g, side="left").astype(jnp.int32)
    ends = jnp.searchsorted(batch, g, side="right").astype(jnp.int32)
    b2 = batch.reshape(NBLK, BR)
    lo = starts[b2[:, 0]]
    hi = ends[b2[:, -1]]
    c0 = lo // CC
    c1 = (hi + CC - 1) // CC
    return jnp.stack([c0, c1], axis=1).reshape(2 * NBLK).astype(jnp.int32)


def kernel(x, batch, bn_gamma, bn_beta, eW1, eb1, eW2, eb2, eW3, eb3,
           dW1, db1, dW2, db2, dW3, db3):
    bounds = _bounds(batch)
    bchunks = batch.reshape(NCHUNK, 1, CC)
    brow = batch.reshape(N, 1)

    h0, ch0 = _bn(x, bn_gamma, bn_beta)
    idx1 = _knn(h0, ch0, bchunks, brow, bounds, D)
    xj1 = _gather(h0, idx1)
    h1, ch1 = _enc(h0, xj1, eW1, eb1, eW2, eb2, eW3, eb3)
    idx2 = _knn(h1, ch1, bchunks, brow, bounds, HID)
    xj2 = _gather(h1, idx2)
    out = _dec(h1, xj2, dW1, db1, dW2, db2, dW3, db3)
    return out
